# baseline (speedup 1.0000x reference)
import functools

import jax
import jax.numpy as jnp
from jax import lax
from jax.experimental import pallas as pl
from jax.experimental.pallas import tpu as pltpu
from jax.experimental.pallas import tpu_sc as plsc

D = 64
DPAD = 128
S = 200
NC, NS = 2, 16
NW = NC * NS
BLK = 128


def _gather_pos_sc(x4, table_rm, pos_rm, batch):
  assert batch == NW * BLK
  n_oct = S // 8
  mesh = plsc.VectorSubcoreMesh(core_axis_name="c", subcore_axis_name="s")

  @functools.partial(
      pl.kernel,
      out_type=jax.ShapeDtypeStruct((S, 8, NW, 8, BLK), jnp.float32),
      mesh=mesh,
      scratch_types=[
          pltpu.VMEM((1, 1, 8, BLK), jnp.int32),
          pltpu.VMEM((1, 1, 8, BLK), jnp.int32),
          pltpu.VMEM((BLK, D), jnp.float32),
          pltpu.VMEM((BLK, D), jnp.float32),
          pltpu.VMEM((1, 8, 1, 8, 129), jnp.float32),
          pltpu.VMEM((1, 8, 1, 8, 129), jnp.float32),
          pltpu.VMEM((S, D), jnp.float32),
          pltpu.SemaphoreType.DMA,
          pltpu.SemaphoreType.DMA,
          pltpu.SemaphoreType.DMA,
      ],
      compiler_params=pltpu.CompilerParams(use_tc_tiling_on_sc=False,
                                           needs_layout_passes=False),
  )
  def k(x_hbm, tab_hbm, pos_hbm, out_hbm, xt0, xt1, g0, g1, t0, t1, posv,
        sem_x, sem_g, sem_w):
    w = lax.axis_index("s") * NC + lax.axis_index("c")
    xts = (xt0, xt1)
    grows = (g0, g1)
    tbufs = (t0, t1)

    lane = lax.iota(jnp.int32, 16)
    z16 = jnp.zeros((16,), jnp.int32)
    dvecs = [lane + 16 * d0 for d0 in range(4)]
    dhis = [dv >> 3 for dv in dvecs]
    dmids = [dv & 7 for dv in dvecs]

    def stage_octet(o, xb):
      return pltpu.async_copy(
          x_hbm.at[pl.ds(o, 1), pl.ds(w, 1), :, :], xts[xb], sem_x)

    def gather(xb, r, gb):
      return pltpu.async_copy(
          tab_hbm.at[xts[xb].at[0, 0, r, :]], grows[gb], sem_g)

    def out_slice(p):
      return out_hbm.at[pl.ds(p, 1), :, pl.ds(w, 1), :, :]

    def tbuf_src(b):
      return tbufs[b].at[:, :, :, :, pl.ds(0, BLK)]

    def position(p, r, xb_cur, xb_nxt, has_next):
      b = r % 2
      pltpu.make_async_copy(
          tab_hbm.at[xts[xb_cur].at[0, 0, r, :]], grows[b], sem_g).wait()
      if r < 7:
        gather(xb_cur, r + 1, 1 - b)
      elif has_next:
        pltpu.make_async_copy(
            x_hbm.at[pl.ds((p + 1) // 8, 1), pl.ds(w, 1), :, :],
            xts[xb_nxt], sem_x).wait()
        gather(xb_nxt, 0, 1 - b)
      @pl.when(p >= 2)
      def _():
        pltpu.make_async_copy(tbuf_src(b), out_slice(p - 2), sem_w).wait()
      sp = jnp.full((16,), p, jnp.int32)
      pvecs = [plsc.load_gather(posv, [sp, dvecs[d0]]) for d0 in range(4)]

      @plsc.parallel_loop(0, BLK, step=1, unroll=4)
      def _(i):
        si = jnp.full((16,), i, jnp.int32)
        for d0 in range(4):
          v = plsc.load_gather(grows[b], [si, dvecs[d0]])
          plsc.store_scatter(tbufs[b], [z16, dhis[d0], z16, dmids[d0], si],
                             v + pvecs[d0])
      pltpu.async_copy(tbuf_src(b), out_slice(p), sem_w)

    pltpu.sync_copy(x_hbm.at[pl.ds(0, 1), pl.ds(w, 1), :, :], xt0)
    pltpu.sync_copy(pos_hbm, posv)
    gather(0, 0, 0)

    def superblock(kk, carry):
      o0 = 2 * kk
      stage_octet(o0 + 1, 1)
      for r in range(8):
        position(o0 * 8 + r, r, 0, 1, True)
      stage_octet(o0 + 2, 0)
      for r in range(8):
        position((o0 + 1) * 8 + r, r, 1, 0, True)
      return carry

    lax.fori_loop(0, (n_oct - 1) // 2, superblock, 0)
    for r in range(8):
      position((n_oct - 1) * 8 + r, r, 0, 1, r < 7)
    pltpu.make_async_copy(tbuf_src(0), out_slice(S - 2), sem_w).wait()
    pltpu.make_async_copy(tbuf_src(1), out_slice(S - 1), sem_w).wait()

  return k(x4, table_rm, pos_rm)


def kernel(x, token_table, pos_table):
  b, s = x.shape
  x4 = (x.astype(jnp.int32).T
        .reshape(S // 8, 8, NW, BLK)
        .transpose(0, 2, 1, 3))
  out5 = _gather_pos_sc(x4, token_table.astype(jnp.float32),
                        pos_table.astype(jnp.float32), b)
  out = (out5.transpose(0, 1, 3, 2, 4)
         .reshape(S, D, b)
         .transpose(2, 0, 1))
  return out


# speedup vs baseline: 1.2709x; 1.2709x over previous
import functools

import jax
import jax.numpy as jnp
from jax import lax
from jax.experimental import pallas as pl
from jax.experimental.pallas import tpu as pltpu
from jax.experimental.pallas import tpu_sc as plsc

D = 64
DPAD = 128
S = 200
NC, NS = 2, 16
NW = NC * NS
BLK = 128


def _gather_pos_sc(x4, table_rm, pos_rm, batch):
  assert batch == NW * BLK
  n_oct = S // 8
  mesh = plsc.VectorSubcoreMesh(core_axis_name="c", subcore_axis_name="s")

  @functools.partial(
      pl.kernel,
      out_type=jax.ShapeDtypeStruct((S, 8, NW, 8, BLK), jnp.float32),
      mesh=mesh,
      scratch_types=[
          pltpu.VMEM((1, 1, 4, 2 * BLK), jnp.int32),
          pltpu.VMEM((1, 1, 4, 2 * BLK), jnp.int32),
          pltpu.VMEM((2 * BLK, D), jnp.float32),
          pltpu.VMEM((2 * BLK, D), jnp.float32),
          pltpu.VMEM((1, 8, 1, 8, 129), jnp.float32),
          pltpu.VMEM((1, 8, 1, 8, 129), jnp.float32),
          pltpu.VMEM((S, D), jnp.float32),
          pltpu.SemaphoreType.DMA,
          pltpu.SemaphoreType.DMA,
          pltpu.SemaphoreType.DMA,
      ],
      compiler_params=pltpu.CompilerParams(use_tc_tiling_on_sc=False,
                                           needs_layout_passes=False),
  )
  def k(x_hbm, tab_hbm, pos_hbm, out_hbm, xt0, xt1, g0, g1, t0, t1, posv,
        sem_x, sem_g, sem_w):
    w = lax.axis_index("s") * NC + lax.axis_index("c")
    xts = (xt0, xt1)
    grows = (g0, g1)
    tbufs = (t0, t1)

    lane = lax.iota(jnp.int32, 16)
    z16 = jnp.zeros((16,), jnp.int32)
    dvecs = [lane + 16 * d0 for d0 in range(4)]
    dhis = [dv >> 3 for dv in dvecs]
    dmids = [dv & 7 for dv in dvecs]

    def stage_octet(o, xb):
      return pltpu.async_copy(
          x_hbm.at[pl.ds(o, 1), pl.ds(w, 1), :, :], xts[xb], sem_x)

    def gather(xb, r, gb):
      return pltpu.async_copy(
          tab_hbm.at[xts[xb].at[0, 0, r // 2, :]], grows[gb], sem_g)

    def out_slice(p):
      return out_hbm.at[pl.ds(p, 1), :, pl.ds(w, 1), :, :]

    def tbuf_src(b):
      return tbufs[b].at[:, :, :, :, pl.ds(0, BLK)]

    def position(p, r, xb_cur, xb_nxt, has_next):
      b = (r // 2) % 2
      if r % 2 == 0:
        pltpu.make_async_copy(
            tab_hbm.at[xts[xb_cur].at[0, 0, r // 2, :]], grows[b],
            sem_g).wait()
        if r < 6:
          gather(xb_cur, r + 2, 1 - b)
        elif has_next:
          pltpu.make_async_copy(
              x_hbm.at[pl.ds((p + 2) // 8, 1), pl.ds(w, 1), :, :],
              xts[xb_nxt], sem_x).wait()
          gather(xb_nxt, 0, 1 - b)
      tb = r % 2
      @pl.when(p >= 2)
      def _():
        pltpu.make_async_copy(tbuf_src(tb), out_slice(p - 2), sem_w).wait()
      sp = jnp.full((16,), p, jnp.int32)
      pvecs = [plsc.load_gather(posv, [sp, dvecs[d0]]) for d0 in range(4)]

      @plsc.parallel_loop(0, BLK, step=1, unroll=4)
      def _(i):
        si = jnp.full((16,), i, jnp.int32)
        gi = si + (r % 2) * BLK
        for d0 in range(4):
          v = plsc.load_gather(grows[b], [gi, dvecs[d0]])
          plsc.store_scatter(tbufs[tb], [z16, dhis[d0], z16, dmids[d0], si],
                             v + pvecs[d0])
      pltpu.async_copy(tbuf_src(tb), out_slice(p), sem_w)

    pltpu.sync_copy(x_hbm.at[pl.ds(0, 1), pl.ds(w, 1), :, :], xt0)
    pltpu.sync_copy(pos_hbm, posv)
    gather(0, 0, 0)

    def superblock(kk, carry):
      o0 = 2 * kk
      stage_octet(o0 + 1, 1)
      for r in range(8):
        position(o0 * 8 + r, r, 0, 1, True)
      stage_octet(o0 + 2, 0)
      for r in range(8):
        position((o0 + 1) * 8 + r, r, 1, 0, True)
      return carry

    lax.fori_loop(0, (n_oct - 1) // 2, superblock, 0)
    for r in range(8):
      position((n_oct - 1) * 8 + r, r, 0, 1, False)
    pltpu.make_async_copy(tbuf_src(0), out_slice(S - 2), sem_w).wait()
    pltpu.make_async_copy(tbuf_src(1), out_slice(S - 1), sem_w).wait()

  return k(x4, table_rm, pos_rm)


def kernel(x, token_table, pos_table):
  b, s = x.shape
  x4 = (x.astype(jnp.int32).T
        .reshape(S // 8, 8, NW, BLK)
        .transpose(0, 2, 1, 3)
        .reshape(S // 8, NW, 4, 2 * BLK))
  out5 = _gather_pos_sc(x4, token_table.astype(jnp.float32),
                        pos_table.astype(jnp.float32), b)
  out = (out5.transpose(0, 1, 3, 2, 4)
         .reshape(S, D, b)
         .transpose(2, 0, 1))
  return out


# speedup vs baseline: 1.3903x; 1.0939x over previous
import functools

import jax
import jax.numpy as jnp
from jax import lax
from jax.experimental import pallas as pl
from jax.experimental.pallas import tpu as pltpu
from jax.experimental.pallas import tpu_sc as plsc

D = 64
DPAD = 128
S = 200
NC, NS = 2, 16
NW = NC * NS
BLK = 128


def _gather_pos_sc(x4, table_rm, pos_rm, batch):
  assert batch == NW * BLK
  n_oct = S // 8
  mesh = plsc.VectorSubcoreMesh(core_axis_name="c", subcore_axis_name="s")

  @functools.partial(
      pl.kernel,
      out_type=jax.ShapeDtypeStruct((S, 8, NW, 8, BLK), jnp.float32),
      mesh=mesh,
      scratch_types=[
          pltpu.VMEM((1, 1, 2, 4 * BLK), jnp.int32),
          pltpu.VMEM((1, 1, 2, 4 * BLK), jnp.int32),
          pltpu.VMEM((4 * BLK, D), jnp.float32),
          pltpu.VMEM((4 * BLK, D), jnp.float32),
          pltpu.VMEM((1, 8, 1, 8, 129), jnp.float32),
          pltpu.VMEM((1, 8, 1, 8, 129), jnp.float32),
          pltpu.VMEM((S, D), jnp.float32),
          pltpu.SemaphoreType.DMA,
          pltpu.SemaphoreType.DMA,
          pltpu.SemaphoreType.DMA,
      ],
      compiler_params=pltpu.CompilerParams(use_tc_tiling_on_sc=False,
                                           needs_layout_passes=False),
  )
  def k(x_hbm, tab_hbm, pos_hbm, out_hbm, xt0, xt1, g0, g1, t0, t1, posv,
        sem_x, sem_g, sem_w):
    w = lax.axis_index("s") * NC + lax.axis_index("c")
    xts = (xt0, xt1)
    grows = (g0, g1)
    tbufs = (t0, t1)

    lane = lax.iota(jnp.int32, 16)
    z16 = jnp.zeros((16,), jnp.int32)
    dvecs = [lane + 16 * d0 for d0 in range(4)]
    dhis = [dv >> 3 for dv in dvecs]
    dmids = [dv & 7 for dv in dvecs]

    def stage_octet(o, xb):
      return pltpu.async_copy(
          x_hbm.at[pl.ds(o, 1), pl.ds(w, 1), :, :], xts[xb], sem_x)

    def gather(xb, r, gb):
      return pltpu.async_copy(
          tab_hbm.at[xts[xb].at[0, 0, r // 4, :]], grows[gb], sem_g)

    def out_slice(p):
      return out_hbm.at[pl.ds(p, 1), :, pl.ds(w, 1), :, :]

    def tbuf_src(b):
      return tbufs[b].at[:, :, :, :, pl.ds(0, BLK)]

    def position(p, r, xb_cur, xb_nxt, has_next):
      b = (r // 4) % 2
      if r % 4 == 0:
        pltpu.make_async_copy(
            tab_hbm.at[xts[xb_cur].at[0, 0, r // 4, :]], grows[b],
            sem_g).wait()
        if r < 4:
          gather(xb_cur, r + 4, 1 - b)
        elif has_next:
          pltpu.make_async_copy(
              x_hbm.at[pl.ds((p + 4) // 8, 1), pl.ds(w, 1), :, :],
              xts[xb_nxt], sem_x).wait()
          gather(xb_nxt, 0, 1 - b)
      tb = r % 2
      @pl.when(p >= 2)
      def _():
        pltpu.make_async_copy(tbuf_src(tb), out_slice(p - 2), sem_w).wait()
      sp = jnp.full((16,), p, jnp.int32)
      pvecs = [plsc.load_gather(posv, [sp, dvecs[d0]]) for d0 in range(4)]

      @plsc.parallel_loop(0, BLK, step=1, unroll=4)
      def _(i):
        si = jnp.full((16,), i, jnp.int32)
        gi = si + (r % 4) * BLK
        for d0 in range(4):
          v = plsc.load_gather(grows[b], [gi, dvecs[d0]])
          plsc.store_scatter(tbufs[tb], [z16, dhis[d0], z16, dmids[d0], si],
                             v + pvecs[d0])
      pltpu.async_copy(tbuf_src(tb), out_slice(p), sem_w)

    pltpu.sync_copy(x_hbm.at[pl.ds(0, 1), pl.ds(w, 1), :, :], xt0)
    pltpu.sync_copy(pos_hbm, posv)
    gather(0, 0, 0)

    def superblock(kk, carry):
      o0 = 2 * kk
      stage_octet(o0 + 1, 1)
      for r in range(8):
        position(o0 * 8 + r, r, 0, 1, True)
      stage_octet(o0 + 2, 0)
      for r in range(8):
        position((o0 + 1) * 8 + r, r, 1, 0, True)
      return carry

    lax.fori_loop(0, (n_oct - 1) // 2, superblock, 0)
    for r in range(8):
      position((n_oct - 1) * 8 + r, r, 0, 1, False)
    pltpu.make_async_copy(tbuf_src(0), out_slice(S - 2), sem_w).wait()
    pltpu.make_async_copy(tbuf_src(1), out_slice(S - 1), sem_w).wait()

  return k(x4, table_rm, pos_rm)


def kernel(x, token_table, pos_table):
  b, s = x.shape
  x4 = (x.astype(jnp.int32).T
        .reshape(S // 8, 8, NW, BLK)
        .transpose(0, 2, 1, 3)
        .reshape(S // 8, NW, 2, 4 * BLK))
  out5 = _gather_pos_sc(x4, token_table.astype(jnp.float32),
                        pos_table.astype(jnp.float32), b)
  out = (out5.transpose(0, 1, 3, 2, 4)
         .reshape(S, D, b)
         .transpose(2, 0, 1))
  return out
